# baseline (speedup 1.0000x reference)
import functools

import jax
import jax.numpy as jnp
from jax import lax
from jax.experimental import pallas as pl
from jax.experimental.pallas import tpu as pltpu
from jax.experimental.pallas import tpu_sc as plsc

N = 10000
DM = 128
G = 64
E = 320000

NT = 16
RPT = 632
NP = NT * RPT
TRASH = N + 8

CHUNK = 128
NCH = 2560
EP = NCH * CHUNK
CPP = 40
PH0 = 4
CPT0 = PH0 * CPP

_MESH = plsc.VectorSubcoreMesh(core_axis_name="c", subcore_axis_name="s")


@functools.partial(
    pl.kernel,
    out_type=jax.ShapeDtypeStruct((2 * NP, DM), jnp.float32),
    mesh=_MESH,
    scratch_types=[
        pltpu.VMEM((NCH // 32, CHUNK), jnp.int32),
        pltpu.VMEM((CHUNK, DM), jnp.float32),
        pltpu.VMEM_SHARED((NP, DM), jnp.float32),
    ],
)
def _deg_kernel(dst_h, ones_h, zeros_h, deg_out, dst_v, ones_v, degsh):
    c = lax.axis_index("c")
    s = lax.axis_index("s")
    w = c * NT + s
    pltpu.sync_copy(dst_h.at[pl.ds(w * (NCH // 32), NCH // 32)], dst_v)
    pltpu.sync_copy(ones_h, ones_v)
    pltpu.sync_copy(zeros_h, degsh.at[pl.ds(s * RPT, RPT)])
    plsc.subcore_barrier()

    def body(j, carry):
        pltpu.sync_copy(ones_v, degsh.at[dst_v.at[j]], add=True)
        return carry

    lax.fori_loop(0, NCH // 32, body, 0)
    plsc.subcore_barrier()
    pltpu.sync_copy(degsh.at[pl.ds(s * RPT, RPT)],
                    deg_out.at[pl.ds(c * NP + s * RPT, RPT)])


@functools.partial(
    pl.kernel,
    out_type=jax.ShapeDtypeStruct((NP, DM), jnp.float32),
    mesh=_MESH,
    scratch_types=[
        pltpu.VMEM((CPP, CHUNK), jnp.int32),
        pltpu.VMEM((CPP, CHUNK), jnp.int32),
        pltpu.VMEM((2, CHUNK, DM), jnp.float32),
        pltpu.VMEM_SHARED((NP, DM), jnp.float32),
        pltpu.SemaphoreType.DMA,
        pltpu.SemaphoreType.DMA,
    ],
)
def _edge_scatter(y_h, src_h, dst_h, zeros_h, z_out, src_v, dst_v, rows_v,
                  zsh, sem_g, sem_s):
    c = lax.axis_index("c")
    s = lax.axis_index("s")
    pltpu.sync_copy(zeros_h, zsh.at[pl.ds(s * RPT, RPT)])
    plsc.subcore_barrier()

    def gather(j, b):
        return pltpu.async_copy(y_h.at[src_v.at[j]], rows_v.at[b], sem_g)

    def scatter(j, b):
        return pltpu.async_copy(rows_v.at[b], zsh.at[dst_v.at[j]], sem_s,
                                add=True)

    def run_phase(p):
        base = s * CPT0 + p * CPP
        pltpu.sync_copy(src_h.at[pl.ds(base, CPP)], src_v)
        pltpu.sync_copy(dst_h.at[pl.ds(base, CPP)], dst_v)
        gather(0, 0).wait()
        scatter(0, 0)
        gather(1, 1)

        def body(j, carry):
            b = lax.rem(j, 2)
            pltpu.make_async_copy(y_h.at[src_v.at[j]], rows_v.at[b],
                                  sem_g).wait()
            scatter(j, b)
            pltpu.make_async_copy(rows_v.at[1 - b], zsh.at[dst_v.at[j]],
                                  sem_s).wait()

            @pl.when(j < CPP - 1)
            def _():
                gather(j + 1, 1 - b)

            return carry

        lax.fori_loop(1, CPP, body, 0)
        pltpu.make_async_copy(rows_v.at[0], zsh.at[dst_v.at[0]],
                              sem_s).wait()

    for p in range(PH0):

        @pl.when(c == 0)
        def _():
            run_phase(p)

    plsc.subcore_barrier()

    @pl.when(c == 0)
    def _():
        pltpu.sync_copy(zsh.at[pl.ds(s * RPT, RPT)],
                        z_out.at[pl.ds(s * RPT, RPT)])


def _tc_embed_body(x_ref, wemb_ref, w0_ref, deg_ref, y_ref, dinv_ref):
    deg = deg_ref[0:N, 0] + deg_ref[NP:NP + N, 0] + 1.0
    dinv = lax.rsqrt(deg)
    w = jnp.dot(wemb_ref[...], w0_ref[...], preferred_element_type=jnp.float32)
    xw = jnp.dot(x_ref[...], w, preferred_element_type=jnp.float32)
    y_ref[0:N, :] = xw * dinv[:, None]
    dinv_ref[0:N] = dinv
    dinv_ref[N:NP] = jnp.zeros((NP - N,), jnp.float32)


def _bn_from(z_ref, y_ref, dinv_ref, b_ref, g_ref, bb_ref):
    dinv = dinv_ref[0:N]
    out = dinv[:, None] * (z_ref[0:N, :] + y_ref[0:N, :]) + b_ref[...]
    mean = jnp.mean(out, axis=0)
    cent = out - mean
    var = jnp.mean(cent * cent, axis=0)
    return cent * lax.rsqrt(var + 1e-5) * g_ref[...] + bb_ref[...], dinv


def _tc_mid_body(z_ref, y_ref, dinv_ref, b_ref, g_ref, bb_ref, w1_ref, y1_ref):
    hn, dinv = _bn_from(z_ref, y_ref, dinv_ref, b_ref, g_ref, bb_ref)
    h1 = jnp.maximum(hn, 0.0)
    xw1 = jnp.dot(h1, w1_ref[...], preferred_element_type=jnp.float32)
    y1_ref[0:N, :] = xw1 * dinv[:, None]


def _tc_tail_body(z_ref, y_ref, dinv_ref, b_ref, g_ref, bb_ref, batch_ref,
                  fcw_ref, fcb_ref, out_ref):
    h2, _ = _bn_from(z_ref, y_ref, dinv_ref, b_ref, g_ref, bb_ref)
    onehot = (lax.broadcasted_iota(jnp.int32, (G, N), 0)
              == batch_ref[...][None, :]).astype(jnp.float32)
    pool = jnp.dot(onehot, h2, preferred_element_type=jnp.float32)
    fw = fcw_ref[...][:, 0]
    out_ref[...] = (jnp.sum(pool * fw[None, :], axis=1, keepdims=True)
                    + fcb_ref[...])


_tc_embed = pl.pallas_call(
    _tc_embed_body,
    out_shape=(jax.ShapeDtypeStruct((NP, DM), jnp.float32),
               jax.ShapeDtypeStruct((NP,), jnp.float32)),
)

_tc_mid = pl.pallas_call(
    _tc_mid_body,
    out_shape=jax.ShapeDtypeStruct((NP, DM), jnp.float32),
)

_tc_tail = pl.pallas_call(
    _tc_tail_body,
    out_shape=jax.ShapeDtypeStruct((G, 1), jnp.float32),
)


def kernel(x, edge_index, batch, W_emb, conv_W0, conv_b0, bn_g0, bn_b0,
           conv_W1, conv_b1, bn_g1, bn_b1, fc_W, fc_b):
    src = edge_index[0]
    dst = edge_index[1]
    pad = EP - E
    src_c = jnp.concatenate([src, jnp.zeros((pad,), jnp.int32)]
                            ).reshape(NCH, CHUNK)
    dst_c = jnp.concatenate([dst, jnp.full((pad,), TRASH, jnp.int32)]
                            ).reshape(NCH, CHUNK)
    zeros128 = jnp.zeros((RPT, DM), jnp.float32)
    ones128 = jnp.ones((CHUNK, DM), jnp.float32)

    deg = _deg_kernel(dst_c, ones128, zeros128)
    y0, dinv = _tc_embed(x, W_emb, conv_W0, deg)
    z0 = _edge_scatter(y0, src_c, dst_c, zeros128)
    y1 = _tc_mid(z0, y0, dinv, conv_b0, bn_g0, bn_b0, conv_W1)
    z1 = _edge_scatter(y1, src_c, dst_c, zeros128)
    return _tc_tail(z1, y1, dinv, conv_b1, bn_g1, bn_b1, batch, fc_W, fc_b)


# speedup vs baseline: 1.3260x; 1.3260x over previous
import functools

import jax
import jax.numpy as jnp
from jax import lax
from jax.experimental import pallas as pl
from jax.experimental.pallas import tpu as pltpu
from jax.experimental.pallas import tpu_sc as plsc

N = 10000
DM = 128
G = 64
E = 320000

NT = 16
RPT = 632
NP = NT * RPT
TRASH = N + 8

CHUNK = 128
NCH = 2560
EP = NCH * CHUNK
CPP = 40
PH0 = 3
PH1 = 1
CPT0 = PH0 * CPP
CPT1 = PH1 * CPP

_MESH = plsc.VectorSubcoreMesh(core_axis_name="c", subcore_axis_name="s")


@functools.partial(
    pl.kernel,
    out_type=jax.ShapeDtypeStruct((2 * NP, DM), jnp.float32),
    mesh=_MESH,
    scratch_types=[
        pltpu.VMEM((NCH // 32, CHUNK), jnp.int32),
        pltpu.VMEM((CHUNK, DM), jnp.float32),
        pltpu.VMEM_SHARED((NP, DM), jnp.float32),
    ],
)
def _deg_kernel(dst_h, ones_h, zeros_h, deg_out, dst_v, ones_v, degsh):
    c = lax.axis_index("c")
    s = lax.axis_index("s")
    w = c * NT + s
    pltpu.sync_copy(dst_h.at[pl.ds(w * (NCH // 32), NCH // 32)], dst_v)
    pltpu.sync_copy(ones_h, ones_v)
    pltpu.sync_copy(zeros_h, degsh.at[pl.ds(s * RPT, RPT)])
    plsc.subcore_barrier()

    def body(j, carry):
        pltpu.sync_copy(ones_v, degsh.at[dst_v.at[j]], add=True)
        return carry

    lax.fori_loop(0, NCH // 32, body, 0)
    plsc.subcore_barrier()
    pltpu.sync_copy(degsh.at[pl.ds(s * RPT, RPT)],
                    deg_out.at[pl.ds(c * NP + s * RPT, RPT)])


@functools.partial(
    pl.kernel,
    out_type=jax.ShapeDtypeStruct((2 * NP, DM), jnp.float32),
    mesh=_MESH,
    scratch_types=[
        pltpu.VMEM((CPP, CHUNK), jnp.int32),
        pltpu.VMEM((CPP, CHUNK), jnp.int32),
        pltpu.VMEM((2, CHUNK, DM), jnp.float32),
        pltpu.VMEM_SHARED((NP, DM), jnp.float32),
        pltpu.SemaphoreType.DMA,
        pltpu.SemaphoreType.DMA,
    ],
)
def _edge_scatter(y_h, src_h, dst_h, zeros_h, z_out, src_v, dst_v, rows_v,
                  zsh, sem_g, sem_s):
    c = lax.axis_index("c")
    s = lax.axis_index("s")
    pltpu.sync_copy(zeros_h, zsh.at[pl.ds(s * RPT, RPT)])
    plsc.subcore_barrier()

    def run_phase(base, pri):
        pltpu.sync_copy(src_h.at[pl.ds(base, CPP)], src_v)
        pltpu.sync_copy(dst_h.at[pl.ds(base, CPP)], dst_v)

        def gather(j, b):
            return pltpu.async_copy(y_h.at[src_v.at[j]], rows_v.at[b],
                                    sem_g, priority=pri)

        def scatter(j, b):
            return pltpu.async_copy(rows_v.at[b], zsh.at[dst_v.at[j]],
                                    sem_s, add=True)

        gather(0, 0).wait()
        scatter(0, 0)
        gather(1, 1)

        def body(j, carry):
            b = lax.rem(j, 2)
            pltpu.make_async_copy(y_h.at[src_v.at[j]], rows_v.at[b],
                                  sem_g).wait()
            scatter(j, b)
            pltpu.make_async_copy(rows_v.at[1 - b], zsh.at[dst_v.at[j]],
                                  sem_s).wait()

            @pl.when(j < CPP - 1)
            def _():
                gather(j + 1, 1 - b)

            return carry

        lax.fori_loop(1, CPP, body, 0)
        pltpu.make_async_copy(rows_v.at[0], zsh.at[dst_v.at[0]],
                              sem_s).wait()

    for p in range(PH0):
        if p < PH1:

            @pl.when(c == 0)
            def _():
                run_phase(s * CPT0 + p * CPP, 0)

            @pl.when(c == 1)
            def _():
                run_phase(16 * CPT0 + s * CPT1 + p * CPP, 1)

        else:

            @pl.when(c == 0)
            def _():
                run_phase(s * CPT0 + p * CPP, 0)

    plsc.subcore_barrier()
    pltpu.sync_copy(zsh.at[pl.ds(s * RPT, RPT)],
                    z_out.at[pl.ds(c * NP + s * RPT, RPT)])


def _tc_embed_body(x_ref, wemb_ref, w0_ref, deg_ref, y_ref, dinv_ref):
    deg = deg_ref[0:N, 0] + deg_ref[NP:NP + N, 0] + 1.0
    dinv = lax.rsqrt(deg)
    w = jnp.dot(wemb_ref[...], w0_ref[...], preferred_element_type=jnp.float32)
    xw = jnp.dot(x_ref[...], w, preferred_element_type=jnp.float32)
    y_ref[0:N, :] = xw * dinv[:, None]
    dinv_ref[0:N] = dinv
    dinv_ref[N:NP] = jnp.zeros((NP - N,), jnp.float32)


def _bn_from(z_ref, y_ref, dinv_ref, b_ref, g_ref, bb_ref):
    dinv = dinv_ref[0:N]
    z = z_ref[0:N, :] + z_ref[NP:NP + N, :]
    out = dinv[:, None] * (z + y_ref[0:N, :]) + b_ref[...]
    mean = jnp.mean(out, axis=0)
    cent = out - mean
    var = jnp.mean(cent * cent, axis=0)
    return cent * lax.rsqrt(var + 1e-5) * g_ref[...] + bb_ref[...], dinv


def _tc_mid_body(z_ref, y_ref, dinv_ref, b_ref, g_ref, bb_ref, w1_ref, y1_ref):
    hn, dinv = _bn_from(z_ref, y_ref, dinv_ref, b_ref, g_ref, bb_ref)
    h1 = jnp.maximum(hn, 0.0)
    xw1 = jnp.dot(h1, w1_ref[...], preferred_element_type=jnp.float32)
    y1_ref[0:N, :] = xw1 * dinv[:, None]


def _tc_tail_body(z_ref, y_ref, dinv_ref, b_ref, g_ref, bb_ref, batch_ref,
                  fcw_ref, fcb_ref, out_ref):
    h2, _ = _bn_from(z_ref, y_ref, dinv_ref, b_ref, g_ref, bb_ref)
    onehot = (lax.broadcasted_iota(jnp.int32, (G, N), 0)
              == batch_ref[...][None, :]).astype(jnp.float32)
    pool = jnp.dot(onehot, h2, preferred_element_type=jnp.float32)
    fw = fcw_ref[...][:, 0]
    out_ref[...] = (jnp.sum(pool * fw[None, :], axis=1, keepdims=True)
                    + fcb_ref[...])


_tc_embed = pl.pallas_call(
    _tc_embed_body,
    out_shape=(jax.ShapeDtypeStruct((NP, DM), jnp.float32),
               jax.ShapeDtypeStruct((NP,), jnp.float32)),
)

_tc_mid = pl.pallas_call(
    _tc_mid_body,
    out_shape=jax.ShapeDtypeStruct((NP, DM), jnp.float32),
)

_tc_tail = pl.pallas_call(
    _tc_tail_body,
    out_shape=jax.ShapeDtypeStruct((G, 1), jnp.float32),
)


def kernel(x, edge_index, batch, W_emb, conv_W0, conv_b0, bn_g0, bn_b0,
           conv_W1, conv_b1, bn_g1, bn_b1, fc_W, fc_b):
    src = edge_index[0]
    dst = edge_index[1]
    pad = EP - E
    src_c = jnp.concatenate([src, jnp.zeros((pad,), jnp.int32)]
                            ).reshape(NCH, CHUNK)
    dst_c = jnp.concatenate([dst, jnp.full((pad,), TRASH, jnp.int32)]
                            ).reshape(NCH, CHUNK)
    zeros128 = jnp.zeros((RPT, DM), jnp.float32)
    ones128 = jnp.ones((CHUNK, DM), jnp.float32)

    deg = _deg_kernel(dst_c, ones128, zeros128)
    y0, dinv = _tc_embed(x, W_emb, conv_W0, deg)
    z0 = _edge_scatter(y0, src_c, dst_c, zeros128)
    y1 = _tc_mid(z0, y0, dinv, conv_b0, bn_g0, bn_b0, conv_W1)
    z1 = _edge_scatter(y1, src_c, dst_c, zeros128)
    return _tc_tail(z1, y1, dinv, conv_b1, bn_g1, bn_b1, batch, fc_W, fc_b)


# speedup vs baseline: 1.3390x; 1.0098x over previous
import functools

import jax
import jax.numpy as jnp
from jax import lax
from jax.experimental import pallas as pl
from jax.experimental.pallas import tpu as pltpu
from jax.experimental.pallas import tpu_sc as plsc

N = 10000
DM = 128
G = 64
E = 320000

NT = 16
RPT = 632
NP = NT * RPT
TRASH = N + 8

CHUNK = 128
NCH = 2560
EP = NCH * CHUNK
CPP = 32
PH0 = 4
PH1 = 1
CPT0 = PH0 * CPP
CPT1 = PH1 * CPP

_MESH = plsc.VectorSubcoreMesh(core_axis_name="c", subcore_axis_name="s")


@functools.partial(
    pl.kernel,
    out_type=jax.ShapeDtypeStruct((2 * NP, DM), jnp.float32),
    mesh=_MESH,
    scratch_types=[
        pltpu.VMEM((NCH // 32, CHUNK), jnp.int32),
        pltpu.VMEM((CHUNK, DM), jnp.float32),
        pltpu.VMEM_SHARED((NP, DM), jnp.float32),
    ],
)
def _deg_kernel(dst_h, ones_h, zeros_h, deg_out, dst_v, ones_v, degsh):
    c = lax.axis_index("c")
    s = lax.axis_index("s")
    w = c * NT + s
    pltpu.sync_copy(dst_h.at[pl.ds(w * (NCH // 32), NCH // 32)], dst_v)
    pltpu.sync_copy(ones_h, ones_v)
    pltpu.sync_copy(zeros_h, degsh.at[pl.ds(s * RPT, RPT)])
    plsc.subcore_barrier()

    def body(j, carry):
        pltpu.sync_copy(ones_v, degsh.at[dst_v.at[j]], add=True)
        return carry

    lax.fori_loop(0, NCH // 32, body, 0)
    plsc.subcore_barrier()
    pltpu.sync_copy(degsh.at[pl.ds(s * RPT, RPT)],
                    deg_out.at[pl.ds(c * NP + s * RPT, RPT)])


@functools.partial(
    pl.kernel,
    out_type=jax.ShapeDtypeStruct((2 * NP, DM), jnp.float32),
    mesh=_MESH,
    scratch_types=[
        pltpu.VMEM((CPP, CHUNK), jnp.int32),
        pltpu.VMEM((CPP, CHUNK), jnp.int32),
        pltpu.VMEM((2, CHUNK, DM), jnp.float32),
        pltpu.VMEM_SHARED((NP, DM), jnp.float32),
        pltpu.SemaphoreType.DMA,
        pltpu.SemaphoreType.DMA,
    ],
)
def _edge_scatter(y_h, src_h, dst_h, zeros_h, z_out, src_v, dst_v, rows_v,
                  zsh, sem_g, sem_s):
    c = lax.axis_index("c")
    s = lax.axis_index("s")
    pltpu.sync_copy(zeros_h, zsh.at[pl.ds(s * RPT, RPT)])
    plsc.subcore_barrier()

    def run_phase(base, pri):
        pltpu.sync_copy(src_h.at[pl.ds(base, CPP)], src_v)
        pltpu.sync_copy(dst_h.at[pl.ds(base, CPP)], dst_v)

        def gather(j, b):
            return pltpu.async_copy(y_h.at[src_v.at[j]], rows_v.at[b],
                                    sem_g, priority=pri)

        def scatter(j, b):
            return pltpu.async_copy(rows_v.at[b], zsh.at[dst_v.at[j]],
                                    sem_s, add=True)

        gather(0, 0).wait()
        scatter(0, 0)
        gather(1, 1)

        def body(j, carry):
            b = lax.rem(j, 2)
            pltpu.make_async_copy(y_h.at[src_v.at[j]], rows_v.at[b],
                                  sem_g).wait()
            scatter(j, b)
            pltpu.make_async_copy(rows_v.at[1 - b], zsh.at[dst_v.at[j]],
                                  sem_s).wait()

            @pl.when(j < CPP - 1)
            def _():
                gather(j + 1, 1 - b)

            return carry

        lax.fori_loop(1, CPP, body, 0)
        pltpu.make_async_copy(rows_v.at[0], zsh.at[dst_v.at[0]],
                              sem_s).wait()

    for p in range(PH0):
        if p < PH1:

            @pl.when(c == 0)
            def _():
                run_phase(s * CPT0 + p * CPP, 0)

            @pl.when(c == 1)
            def _():
                run_phase(16 * CPT0 + s * CPT1 + p * CPP, 1)

        else:

            @pl.when(c == 0)
            def _():
                run_phase(s * CPT0 + p * CPP, 0)

    plsc.subcore_barrier()
    pltpu.sync_copy(zsh.at[pl.ds(s * RPT, RPT)],
                    z_out.at[pl.ds(c * NP + s * RPT, RPT)])


def _tc_embed_body(x_ref, wemb_ref, w0_ref, deg_ref, y_ref, dinv_ref):
    deg = deg_ref[0:N, 0] + deg_ref[NP:NP + N, 0] + 1.0
    dinv = lax.rsqrt(deg)
    w = jnp.dot(wemb_ref[...], w0_ref[...], preferred_element_type=jnp.float32)
    xw = jnp.dot(x_ref[...], w, preferred_element_type=jnp.float32)
    y_ref[0:N, :] = xw * dinv[:, None]
    dinv_ref[0:N] = dinv
    dinv_ref[N:NP] = jnp.zeros((NP - N,), jnp.float32)


def _bn_from(z_ref, y_ref, dinv_ref, b_ref, g_ref, bb_ref):
    dinv = dinv_ref[0:N]
    z = z_ref[0:N, :] + z_ref[NP:NP + N, :]
    out = dinv[:, None] * (z + y_ref[0:N, :]) + b_ref[...]
    mean = jnp.mean(out, axis=0)
    cent = out - mean
    var = jnp.mean(cent * cent, axis=0)
    return cent * lax.rsqrt(var + 1e-5) * g_ref[...] + bb_ref[...], dinv


def _tc_mid_body(z_ref, y_ref, dinv_ref, b_ref, g_ref, bb_ref, w1_ref, y1_ref):
    hn, dinv = _bn_from(z_ref, y_ref, dinv_ref, b_ref, g_ref, bb_ref)
    h1 = jnp.maximum(hn, 0.0)
    xw1 = jnp.dot(h1, w1_ref[...], preferred_element_type=jnp.float32)
    y1_ref[0:N, :] = xw1 * dinv[:, None]


def _tc_tail_body(z_ref, y_ref, dinv_ref, b_ref, g_ref, bb_ref, batch_ref,
                  fcw_ref, fcb_ref, out_ref):
    h2, _ = _bn_from(z_ref, y_ref, dinv_ref, b_ref, g_ref, bb_ref)
    onehot = (lax.broadcasted_iota(jnp.int32, (G, N), 0)
              == batch_ref[...][None, :]).astype(jnp.float32)
    pool = jnp.dot(onehot, h2, preferred_element_type=jnp.float32)
    fw = fcw_ref[...][:, 0]
    out_ref[...] = (jnp.sum(pool * fw[None, :], axis=1, keepdims=True)
                    + fcb_ref[...])


_tc_embed = pl.pallas_call(
    _tc_embed_body,
    out_shape=(jax.ShapeDtypeStruct((NP, DM), jnp.float32),
               jax.ShapeDtypeStruct((NP,), jnp.float32)),
)

_tc_mid = pl.pallas_call(
    _tc_mid_body,
    out_shape=jax.ShapeDtypeStruct((NP, DM), jnp.float32),
)

_tc_tail = pl.pallas_call(
    _tc_tail_body,
    out_shape=jax.ShapeDtypeStruct((G, 1), jnp.float32),
)


def kernel(x, edge_index, batch, W_emb, conv_W0, conv_b0, bn_g0, bn_b0,
           conv_W1, conv_b1, bn_g1, bn_b1, fc_W, fc_b):
    src = edge_index[0]
    dst = edge_index[1]
    pad = EP - E
    src_c = jnp.concatenate([src, jnp.zeros((pad,), jnp.int32)]
                            ).reshape(NCH, CHUNK)
    dst_c = jnp.concatenate([dst, jnp.full((pad,), TRASH, jnp.int32)]
                            ).reshape(NCH, CHUNK)
    zeros128 = jnp.zeros((RPT, DM), jnp.float32)
    ones128 = jnp.ones((CHUNK, DM), jnp.float32)

    deg = _deg_kernel(dst_c, ones128, zeros128)
    y0, dinv = _tc_embed(x, W_emb, conv_W0, deg)
    z0 = _edge_scatter(y0, src_c, dst_c, zeros128)
    y1 = _tc_mid(z0, y0, dinv, conv_b0, bn_g0, bn_b0, conv_W1)
    z1 = _edge_scatter(y1, src_c, dst_c, zeros128)
    return _tc_tail(z1, y1, dinv, conv_b1, bn_g1, bn_b1, batch, fc_W, fc_b)
